# baseline (speedup 1.0000x reference)
import dataclasses
import functools
import jax
import jax.numpy as jnp
from jax import lax
from jax.experimental import pallas as pl
from jax.experimental.pallas import tpu as pltpu
from jax.experimental.pallas import tpu_sc as plsc

B = 1024
DIM = 16
QUEUE = 100000
TEMP = 0.07
K = 20
CLASSES = 1000

CHUNK = 4096
NCHUNK = 25
QPAD = CHUNK * NCHUNK
GSZ = 128
GPC = CHUNK // GSZ
NG = QPAD // GSZ

NW = 32
RPW = B // NW
CAP = 32
FLAGCAP = 832
CCAP = 2048
CPAD = CCAP + 16
OPAD = 1024


def _dist_body(x_ref, mem_ref, dist_ref, gmax_ref):
    c = pl.program_id(0)
    x = x_ref[...]
    nrm = jnp.sqrt(jnp.sum(x * x, axis=1, keepdims=True))
    xn = x / jnp.maximum(nrm, 1e-12)
    mem = mem_ref[...]
    scores = lax.dot_general(
        xn, mem, (((1,), (1,)), ((), ())),
        preferred_element_type=jnp.float32)
    gcol = c * CHUNK + lax.broadcasted_iota(jnp.int32, (B, CHUNK), 1)
    scores = jnp.where(gcol < QUEUE, scores, -jnp.inf)
    dist_ref[...] = scores
    scores_t = lax.dot_general(
        mem, xn, (((1,), (1,)), ((), ())),
        preferred_element_type=jnp.float32)
    grow = c * CHUNK + lax.broadcasted_iota(jnp.int32, (CHUNK, B), 0)
    scores_t = jnp.where(grow < QUEUE, scores_t, -jnp.inf)
    gmax_ref[...] = jnp.max(scores_t.reshape(GPC, GSZ, B), axis=1)


def _run_dist(x, memory):
    return pl.pallas_call(
        _dist_body,
        grid=(NCHUNK,),
        in_specs=[
            pl.BlockSpec((B, DIM), lambda c: (0, 0)),
            pl.BlockSpec((CHUNK, DIM), lambda c: (c, 0)),
        ],
        out_specs=[
            pl.BlockSpec((B, CHUNK), lambda c: (0, c)),
            pl.BlockSpec((GPC, B), lambda c: (c, 0)),
        ],
        out_shape=[
            jax.ShapeDtypeStruct((B, QPAD), jnp.float32),
            jax.ShapeDtypeStruct((NG, B), jnp.float32),
        ],
        compiler_params=pltpu.CompilerParams(
            dimension_semantics=("arbitrary",)),
    )(x, memory)


def _thresh_body(g_ref, t_ref):
    def step(_k, carry):
        g, _t = carry
        m = jnp.max(g, axis=0, keepdims=True)
        return jnp.where(g == m, -jnp.inf, g), m

    _, t = lax.fori_loop(
        0, K, step, (g_ref[...], jnp.zeros((1, B), jnp.float32)))
    t_ref[...] = t


def _run_thresh(gmax_t):
    return pl.pallas_call(
        _thresh_body,
        out_shape=jax.ShapeDtypeStruct((1, B), jnp.float32),
    )(gmax_t)


def _sc_refine(dist2, gmax, t, memory_label):
    mesh = plsc.VectorSubcoreMesh(core_axis_name="c", subcore_axis_name="s")
    cp = pltpu.CompilerParams()
    if "needs_layout_passes" in pltpu.CompilerParams.__dataclass_fields__:
        cp = dataclasses.replace(cp, needs_layout_passes=False)

    @functools.partial(
        pl.kernel,
        out_type=jax.ShapeDtypeStruct((B, OPAD), jnp.float32),
        mesh=mesh,
        compiler_params=cp,
        scratch_types=[
            pltpu.VMEM((QUEUE,), jnp.int32),
            pltpu.VMEM((NG,), jnp.float32),
            pltpu.VMEM((FLAGCAP,), jnp.int32),
            pltpu.VMEM((CAP * GSZ,), jnp.float32),
            pltpu.VMEM((CPAD,), jnp.float32),
            pltpu.VMEM((CPAD,), jnp.int32),
            pltpu.VMEM((48,), jnp.float32),
            pltpu.VMEM((48,), jnp.int32),
            pltpu.VMEM((32,), jnp.int32),
            pltpu.VMEM((32,), jnp.float32),
            pltpu.VMEM((OPAD,), jnp.float32),
            pltpu.VMEM((48,), jnp.float32),
            pltpu.SemaphoreType.DMA,
        ],
    )
    def kb(dist_hbm, gmax_hbm, t_hbm, lab_hbm, out_hbm,
           labv, gv, flagv, dbuf, candv, candiv, wv, wiv, wlv, wwv,
           rowv, tv, sem):
        wid = lax.axis_index("s") * 2 + lax.axis_index("c")
        base = wid * RPW
        pltpu.async_copy(lab_hbm, labv, sem).wait()
        pltpu.async_copy(
            t_hbm.at[pl.ds(base, RPW)], tv.at[pl.ds(0, RPW)], sem).wait()
        iota16 = lax.iota(jnp.int32, 16)

        @pl.loop(0, RPW)
        def _row(rl):
            r = base + rl
            pltpu.async_copy(gmax_hbm.at[r], gv, sem).wait()
            t = tv[pl.ds(rl, 16)][0] - 1e-3

            def fbody(j, cur):
                gvj = gv[pl.ds(j * 16, 16)]
                msk = gvj >= t
                ranks = plsc.cumsum(msk.astype(jnp.int32))
                pos = cur + ranks - 1
                ids = j * 16 + iota16
                plsc.store_scatter(flagv, [pos], ids, mask=msk)
                return cur + jnp.sum(msk.astype(jnp.int32))

            nflag = lax.fori_loop(0, NG // 16, fbody, jnp.int32(0))
            nb = (nflag + (CAP - 1)) // CAP

            def bbody(b, ccur):
                ilim = jnp.minimum(nflag - b * CAP, CAP)

                def issue(i, _):
                    g = flagv[pl.ds(b * CAP + i, 16)][0]
                    pltpu.async_copy(
                        dist_hbm.at[r, pl.ds(g * GSZ, GSZ)],
                        dbuf.at[pl.ds(i * GSZ, GSZ)], sem)
                    return 0

                lax.fori_loop(0, ilim, issue, jnp.int32(0))

                def drain(i, _):
                    pltpu.make_async_copy(
                        dist_hbm.at[r, pl.ds(0, GSZ)],
                        dbuf.at[pl.ds(i * GSZ, GSZ)], sem).wait()
                    return 0

                lax.fori_loop(0, ilim, drain, jnp.int32(0))

                def ibody(i, ccur2):
                    colbase = flagv[pl.ds(b * CAP + i, 16)][0] * GSZ

                    def jbody(j2, ccur3):
                        v = dbuf[pl.ds(i * GSZ + j2 * 16, 16)]
                        msk2 = v >= t
                        ranks2 = plsc.cumsum(msk2.astype(jnp.int32))
                        pos2 = jnp.minimum(ccur3 + ranks2 - 1, CCAP - 1)
                        colv = colbase + j2 * 16 + iota16
                        plsc.store_scatter(candv, [pos2], v, mask=msk2)
                        plsc.store_scatter(candiv, [pos2], colv, mask=msk2)
                        return ccur3 + jnp.sum(msk2.astype(jnp.int32))

                    return lax.fori_loop(0, GSZ // 16, jbody, ccur2)

                return lax.fori_loop(0, ilim, ibody, ccur)

            ncand = lax.fori_loop(0, nb, bbody, jnp.int32(0))
            ncand = jnp.minimum(ncand, CCAP)
            candv[pl.ds(ncand, 16)] = jnp.full((16,), -jnp.inf, jnp.float32)
            nv = (ncand + 15) // 16

            for jj in (0, 16, 32):
                wv[pl.ds(jj, 16)] = jnp.full((16,), -jnp.inf, jnp.float32)
                wiv[pl.ds(jj, 16)] = jnp.zeros((16,), jnp.int32)

            def kbody(k, carry):
                def smax(v2, mv):
                    return jnp.maximum(mv, candv[pl.ds(v2 * 16, 16)])

                mv = lax.fori_loop(
                    0, nv, smax, jnp.full((16,), -jnp.inf, jnp.float32))
                m = jnp.max(mv)

                def sfind(v2, p):
                    lv = candv[pl.ds(v2 * 16, 16)]
                    pv = jnp.where(lv == m, v2 * 16 + iota16,
                                   jnp.int32(2 ** 30))
                    return jnp.minimum(p, jnp.min(pv))

                pos = lax.fori_loop(0, nv, sfind, jnp.int32(2 ** 30))
                lane0 = iota16 == 0
                wwin = wv[pl.ds(k, 16)]
                wv[pl.ds(k, 16)] = jnp.where(lane0, m, wwin)
                ci = candiv[pl.ds(pos, 16)][0]
                iwin = wiv[pl.ds(k, 16)]
                wiv[pl.ds(k, 16)] = jnp.where(lane0, ci, iwin)
                cwin = candv[pl.ds(pos, 16)]
                candv[pl.ds(pos, 16)] = jnp.where(
                    lane0, -jnp.inf, cwin)
                return carry

            lax.fori_loop(0, K, kbody, jnp.int32(0))

            for jj in (0, 16):
                iv = wiv[pl.ds(jj, 16)]
                wlv[pl.ds(jj, 16)] = plsc.load_gather(labv, [iv])

            m0 = wv[pl.ds(0, 16)][0]
            e0 = jnp.exp((wv[pl.ds(0, 16)] - m0) * (1.0 / TEMP))
            e1 = jnp.exp((wv[pl.ds(16, 16)] - m0) * (1.0 / TEMP))
            s = jnp.sum(e0) + jnp.sum(e1)
            wwv[pl.ds(0, 16)] = e0 / s
            wwv[pl.ds(16, 16)] = e1 / s

            @pl.loop(0, OPAD, step=16)
            def _(p):
                rowv[pl.ds(p, 16)] = jnp.zeros((16,), jnp.float32)

            lv0 = wlv[pl.ds(0, 16)]
            lv1 = wlv[pl.ds(16, 16)]
            wt0 = wwv[pl.ds(0, 16)]
            wt1 = wwv[pl.ds(16, 16)]
            for j in range(K):
                lbl = lv0[j] if j < 16 else lv1[j - 16]
                wj = wt0[j] if j < 16 else wt1[j - 16]
                rwin = rowv[pl.ds(lbl, 16)]
                rowv[pl.ds(lbl, 16)] = rwin + jnp.where(
                    iota16 == 0, wj, 0.0)

            @pl.loop(0, OPAD, step=16)
            def _(p):
                rowv[pl.ds(p, 16)] = jnp.minimum(
                    rowv[pl.ds(p, 16)] + 1e-5, 1.0)

            pltpu.sync_copy(rowv, out_hbm.at[r])

    return kb(dist2, gmax, t, memory_label)


def kernel(x, memory, memory_label):
    dist, gmax_t = _run_dist(x, memory)
    t = _run_thresh(gmax_t)
    gmax = gmax_t.T
    out = _sc_refine(dist, gmax, t.reshape(B), memory_label)
    return out[:, :CLASSES]


# speedup vs baseline: 1.0067x; 1.0067x over previous
import dataclasses
import functools
import jax
import jax.numpy as jnp
from jax import lax
from jax.experimental import pallas as pl
from jax.experimental.pallas import tpu as pltpu
from jax.experimental.pallas import tpu_sc as plsc

B = 1024
DIM = 16
QUEUE = 100000
TEMP = 0.07
K = 20
CLASSES = 1000

CHUNK = 2048
NCHUNK = 49
QPAD = CHUNK * NCHUNK
GSZ = 128
GPC = CHUNK // GSZ
NG = QPAD // GSZ

NW = 32
RPW = B // NW
CAP = 32
FLAGCAP = 832
CCAP = 2048
CPAD = CCAP + 16
OPAD = 1024


def _dist_body(x_ref, mem_ref, dist_ref, gmax_ref):
    c = pl.program_id(0)
    x = x_ref[...]
    nrm = jnp.sqrt(jnp.sum(x * x, axis=1, keepdims=True))
    xn = x / jnp.maximum(nrm, 1e-12)
    mem = mem_ref[...]
    scores = lax.dot_general(
        xn, mem, (((1,), (1,)), ((), ())),
        preferred_element_type=jnp.float32)
    gcol = c * CHUNK + lax.broadcasted_iota(jnp.int32, (B, CHUNK), 1)
    scores = jnp.where(gcol < QUEUE, scores, -jnp.inf)
    dist_ref[...] = scores
    scores_t = lax.dot_general(
        mem, xn, (((1,), (1,)), ((), ())),
        preferred_element_type=jnp.float32)
    grow = c * CHUNK + lax.broadcasted_iota(jnp.int32, (CHUNK, B), 0)
    scores_t = jnp.where(grow < QUEUE, scores_t, -jnp.inf)
    gmax_ref[...] = jnp.max(scores_t.reshape(GPC, GSZ, B), axis=1)


def _run_dist(x, memory):
    return pl.pallas_call(
        _dist_body,
        grid=(NCHUNK,),
        in_specs=[
            pl.BlockSpec((B, DIM), lambda c: (0, 0)),
            pl.BlockSpec((CHUNK, DIM), lambda c: (c, 0)),
        ],
        out_specs=[
            pl.BlockSpec((B, CHUNK), lambda c: (0, c)),
            pl.BlockSpec((GPC, B), lambda c: (c, 0)),
        ],
        out_shape=[
            jax.ShapeDtypeStruct((B, QPAD), jnp.float32),
            jax.ShapeDtypeStruct((NG, B), jnp.float32),
        ],
        compiler_params=pltpu.CompilerParams(
            dimension_semantics=("arbitrary",)),
    )(x, memory)


def _thresh_body(g_ref, t_ref):
    def step(_k, carry):
        g, _t = carry
        m = jnp.max(g, axis=0, keepdims=True)
        return jnp.where(g == m, -jnp.inf, g), m

    _, t = lax.fori_loop(
        0, K, step, (g_ref[...], jnp.zeros((1, B), jnp.float32)))
    t_ref[...] = t


def _run_thresh(gmax_t):
    return pl.pallas_call(
        _thresh_body,
        out_shape=jax.ShapeDtypeStruct((1, B), jnp.float32),
    )(gmax_t)


def _sc_refine(dist2, gmax, t, memory_label):
    mesh = plsc.VectorSubcoreMesh(core_axis_name="c", subcore_axis_name="s")
    cp = pltpu.CompilerParams()
    if "needs_layout_passes" in pltpu.CompilerParams.__dataclass_fields__:
        cp = dataclasses.replace(cp, needs_layout_passes=False)

    @functools.partial(
        pl.kernel,
        out_type=jax.ShapeDtypeStruct((B, OPAD), jnp.float32),
        mesh=mesh,
        compiler_params=cp,
        scratch_types=[
            pltpu.VMEM((QUEUE,), jnp.int32),
            pltpu.VMEM((NG,), jnp.float32),
            pltpu.VMEM((FLAGCAP,), jnp.int32),
            pltpu.VMEM((CAP * GSZ,), jnp.float32),
            pltpu.VMEM((CPAD,), jnp.float32),
            pltpu.VMEM((CPAD,), jnp.int32),
            pltpu.VMEM((48,), jnp.float32),
            pltpu.VMEM((48,), jnp.int32),
            pltpu.VMEM((32,), jnp.int32),
            pltpu.VMEM((32,), jnp.float32),
            pltpu.VMEM((OPAD,), jnp.float32),
            pltpu.VMEM((48,), jnp.float32),
            pltpu.SemaphoreType.DMA,
        ],
    )
    def kb(dist_hbm, gmax_hbm, t_hbm, lab_hbm, out_hbm,
           labv, gv, flagv, dbuf, candv, candiv, wv, wiv, wlv, wwv,
           rowv, tv, sem):
        wid = lax.axis_index("s") * 2 + lax.axis_index("c")
        base = wid * RPW
        pltpu.async_copy(lab_hbm, labv, sem).wait()
        pltpu.async_copy(
            t_hbm.at[pl.ds(base, RPW)], tv.at[pl.ds(0, RPW)], sem).wait()
        iota16 = lax.iota(jnp.int32, 16)

        @pl.loop(0, RPW)
        def _row(rl):
            r = base + rl
            pltpu.async_copy(gmax_hbm.at[r], gv, sem).wait()
            t = tv[pl.ds(rl, 16)][0] - 1e-3

            def fbody(j, cur):
                gvj = gv[pl.ds(j * 16, 16)]
                msk = gvj >= t
                ranks = plsc.cumsum(msk.astype(jnp.int32))
                pos = cur + ranks - 1
                ids = j * 16 + iota16
                plsc.store_scatter(flagv, [pos], ids, mask=msk)
                return cur + jnp.sum(msk.astype(jnp.int32))

            nflag = lax.fori_loop(0, NG // 16, fbody, jnp.int32(0))
            nb = (nflag + (CAP - 1)) // CAP

            def bbody(b, ccur):
                ilim = jnp.minimum(nflag - b * CAP, CAP)

                def issue(i, _):
                    g = flagv[pl.ds(b * CAP + i, 16)][0]
                    pltpu.async_copy(
                        dist_hbm.at[r, pl.ds(g * GSZ, GSZ)],
                        dbuf.at[pl.ds(i * GSZ, GSZ)], sem)
                    return 0

                lax.fori_loop(0, ilim, issue, jnp.int32(0))

                def drain(i, _):
                    pltpu.make_async_copy(
                        dist_hbm.at[r, pl.ds(0, GSZ)],
                        dbuf.at[pl.ds(i * GSZ, GSZ)], sem).wait()
                    return 0

                lax.fori_loop(0, ilim, drain, jnp.int32(0))

                def ibody(i, ccur2):
                    colbase = flagv[pl.ds(b * CAP + i, 16)][0] * GSZ

                    def jbody(j2, ccur3):
                        v = dbuf[pl.ds(i * GSZ + j2 * 16, 16)]
                        msk2 = v >= t
                        ranks2 = plsc.cumsum(msk2.astype(jnp.int32))
                        pos2 = jnp.minimum(ccur3 + ranks2 - 1, CCAP - 1)
                        colv = colbase + j2 * 16 + iota16
                        plsc.store_scatter(candv, [pos2], v, mask=msk2)
                        plsc.store_scatter(candiv, [pos2], colv, mask=msk2)
                        return ccur3 + jnp.sum(msk2.astype(jnp.int32))

                    return lax.fori_loop(0, GSZ // 16, jbody, ccur2)

                return lax.fori_loop(0, ilim, ibody, ccur)

            ncand = lax.fori_loop(0, nb, bbody, jnp.int32(0))
            ncand = jnp.minimum(ncand, CCAP)
            candv[pl.ds(ncand, 16)] = jnp.full((16,), -jnp.inf, jnp.float32)
            nv = (ncand + 15) // 16

            for jj in (0, 16, 32):
                wv[pl.ds(jj, 16)] = jnp.full((16,), -jnp.inf, jnp.float32)
                wiv[pl.ds(jj, 16)] = jnp.zeros((16,), jnp.int32)

            def kbody(k, carry):
                def smax(v2, mv):
                    return jnp.maximum(mv, candv[pl.ds(v2 * 16, 16)])

                mv = lax.fori_loop(
                    0, nv, smax, jnp.full((16,), -jnp.inf, jnp.float32))
                m = jnp.max(mv)

                def sfind(v2, p):
                    lv = candv[pl.ds(v2 * 16, 16)]
                    pv = jnp.where(lv == m, v2 * 16 + iota16,
                                   jnp.int32(2 ** 30))
                    return jnp.minimum(p, jnp.min(pv))

                pos = lax.fori_loop(0, nv, sfind, jnp.int32(2 ** 30))
                lane0 = iota16 == 0
                wwin = wv[pl.ds(k, 16)]
                wv[pl.ds(k, 16)] = jnp.where(lane0, m, wwin)
                ci = candiv[pl.ds(pos, 16)][0]
                iwin = wiv[pl.ds(k, 16)]
                wiv[pl.ds(k, 16)] = jnp.where(lane0, ci, iwin)
                cwin = candv[pl.ds(pos, 16)]
                candv[pl.ds(pos, 16)] = jnp.where(
                    lane0, -jnp.inf, cwin)
                return carry

            lax.fori_loop(0, K, kbody, jnp.int32(0))

            for jj in (0, 16):
                iv = wiv[pl.ds(jj, 16)]
                wlv[pl.ds(jj, 16)] = plsc.load_gather(labv, [iv])

            m0 = wv[pl.ds(0, 16)][0]
            e0 = jnp.exp((wv[pl.ds(0, 16)] - m0) * (1.0 / TEMP))
            e1 = jnp.exp((wv[pl.ds(16, 16)] - m0) * (1.0 / TEMP))
            s = jnp.sum(e0) + jnp.sum(e1)
            wwv[pl.ds(0, 16)] = e0 / s
            wwv[pl.ds(16, 16)] = e1 / s

            @pl.loop(0, OPAD, step=16)
            def _(p):
                rowv[pl.ds(p, 16)] = jnp.zeros((16,), jnp.float32)

            lv0 = wlv[pl.ds(0, 16)]
            lv1 = wlv[pl.ds(16, 16)]
            wt0 = wwv[pl.ds(0, 16)]
            wt1 = wwv[pl.ds(16, 16)]
            for j in range(K):
                lbl = lv0[j] if j < 16 else lv1[j - 16]
                wj = wt0[j] if j < 16 else wt1[j - 16]
                rwin = rowv[pl.ds(lbl, 16)]
                rowv[pl.ds(lbl, 16)] = rwin + jnp.where(
                    iota16 == 0, wj, 0.0)

            @pl.loop(0, OPAD, step=16)
            def _(p):
                rowv[pl.ds(p, 16)] = jnp.minimum(
                    rowv[pl.ds(p, 16)] + 1e-5, 1.0)

            pltpu.sync_copy(rowv, out_hbm.at[r])

    return kb(dist2, gmax, t, memory_label)


def kernel(x, memory, memory_label):
    dist, gmax_t = _run_dist(x, memory)
    t = _run_thresh(gmax_t)
    gmax = gmax_t.T
    out = _sc_refine(dist, gmax, t.reshape(B), memory_label)
    return out[:, :CLASSES]
